# Initial kernel scaffold; baseline (speedup 1.0000x reference)
import jax
import jax.numpy as jnp
from jax.experimental import pallas as pl


def kernel(hour, day, recency, hour_table, day_table, W, b):
    raise NotImplementedError("write your pallas kernel here")


# speedup vs baseline: 16.4628x; 16.4628x over previous
import jax
import jax.numpy as jnp
from jax.experimental import pallas as pl
from jax.experimental.pallas import tpu as pltpu

_B, _S, _D = 16384, 200, 128
_N = _B * _S
_HALF = _D // 2
_BR = 16
_NR = _N // 128
_GRID = _NR // _BR


def _tc_body(h_ref, d_ref, r_ref, t3_ref, o_ref):
    h = h_ref[...]
    d = d_ref[...]
    r = r_ref[...]
    io = jax.lax.broadcasted_iota(jnp.int32, (_BR, 128, 32), 2)
    hot = (io == h[:, :, None]) | (io == (d[:, :, None] + 24))
    m = jnp.where(io == 31, r[:, :, None], hot.astype(jnp.float32))
    m2 = m.reshape(_BR * 128, 32)
    o_ref[...] = jnp.dot(m2, t3_ref[...], preferred_element_type=jnp.float32)


def kernel(hour, day, recency, hour_table, day_table, W, b):
    h2 = hour.reshape(_NR, 128)
    d2 = day.reshape(_NR, 128)
    r2 = recency.reshape(_NR, 128)
    t3 = jnp.zeros((32, _D), jnp.float32)
    t3 = t3.at[0:24, 0:_HALF].set(hour_table + b[:_HALF])
    t3 = t3.at[24:31, _HALF:].set(day_table + b[_HALF:])
    t3 = t3.at[31, :].set(W[0])

    out = pl.pallas_call(
        _tc_body,
        grid=(_GRID,),
        in_specs=[
            pl.BlockSpec((_BR, 128), lambda i: (i, 0)),
            pl.BlockSpec((_BR, 128), lambda i: (i, 0)),
            pl.BlockSpec((_BR, 128), lambda i: (i, 0)),
            pl.BlockSpec((32, _D), lambda i: (0, 0)),
        ],
        out_specs=pl.BlockSpec((_BR * 128, _D), lambda i: (i, 0)),
        out_shape=jax.ShapeDtypeStruct((_N, _D), jnp.float32),
        compiler_params=pltpu.CompilerParams(
            dimension_semantics=("arbitrary",),
        ),
    )(h2, d2, r2, t3)
    return out.reshape(_B, _S, _D)
